# baseline (speedup 1.0000x reference)
import functools

import jax
import jax.numpy as jnp
from jax import lax
from jax.experimental import pallas as pl
from jax.experimental.pallas import tpu as pltpu
from jax.experimental.pallas import tpu_sc as plsc

BATCH = 16384
CODE = 64
CHUNK = 128
NW = 32
CPW = BATCH // (NW * CHUNK)
BPW = BATCH // NW

_mesh = plsc.VectorSubcoreMesh(core_axis_name="c", subcore_axis_name="s")


@functools.partial(
    pl.kernel,
    mesh=_mesh,
    out_type=(
        jax.ShapeDtypeStruct((BATCH, CODE), jnp.float32),
        jax.ShapeDtypeStruct((BATCH, CODE), jnp.float32),
    ),
    scratch_types=[
        pltpu.VMEM((CPW, CHUNK), jnp.int32),
        pltpu.VMEM((BPW, CODE), jnp.float32),
        pltpu.VMEM((BPW, CODE), jnp.float32),
        pltpu.SemaphoreType.DMA,
    ],
    compiler_params=pltpu.CompilerParams(use_tc_tiling_on_sc=False),
)
def _gather2(idx_hbm, ws_hbm, wa_hbm, out_s, out_a, idx_v, rows_s, rows_a, sem):
    wid = lax.axis_index("s") * 2 + lax.axis_index("c")
    pltpu.sync_copy(idx_hbm.at[pl.ds(wid * CPW, CPW)], idx_v)
    copies = []
    for j in range(CPW):
        copies.append(pltpu.async_copy(
            ws_hbm.at[idx_v.at[j]], rows_s.at[pl.ds(j * CHUNK, CHUNK)], sem))
        copies.append(pltpu.async_copy(
            wa_hbm.at[idx_v.at[j]], rows_a.at[pl.ds(j * CHUNK, CHUNK)], sem))
    for c in copies:
        c.wait()
    base = wid * BPW
    pltpu.sync_copy(rows_s, out_s.at[pl.ds(base, BPW)])
    pltpu.sync_copy(rows_a, out_a.at[pl.ds(base, BPW)])


def kernel(instance_ids, W_shape, W_appearance):
    idx = instance_ids.astype(jnp.int32).reshape(NW * CPW, CHUNK)
    return _gather2(idx, W_shape, W_appearance)


# speedup vs baseline: 1.5802x; 1.5802x over previous
import functools

import jax
import jax.numpy as jnp
from jax import lax
from jax.experimental import pallas as pl
from jax.experimental.pallas import tpu as pltpu
from jax.experimental.pallas import tpu_sc as plsc

BATCH = 16384
CODE = 64
NW = 32
BPW = BATCH // NW
HALF = BPW // 2
K = 16

_mesh = plsc.VectorSubcoreMesh(core_axis_name="c", subcore_axis_name="s")


@functools.partial(
    pl.kernel,
    mesh=_mesh,
    out_type=(
        jax.ShapeDtypeStruct((BATCH, CODE), jnp.float32),
        jax.ShapeDtypeStruct((BATCH, CODE), jnp.float32),
    ),
    scratch_types=[
        pltpu.VMEM((BPW,), jnp.int32),
        pltpu.VMEM((HALF, CODE), jnp.float32),
        pltpu.VMEM((HALF, CODE), jnp.float32),
        pltpu.SemaphoreType.DMA,
    ],
)
def _gather2(idx_hbm, ws_hbm, wa_hbm, out_s, out_a, idx_v, rows_s, rows_a, sem):
    wid = lax.axis_index("s") * 2 + lax.axis_index("c")
    base = wid * BPW
    pltpu.sync_copy(idx_hbm.at[pl.ds(base, BPW)], idx_v)

    for h in range(2):
        off = h * HALF

        def chunk_body(c, carry):
            vec = idx_v[pl.ds(off + c * K, K)]
            for j in range(K):
                r = vec[j]
                i = c * K + j
                pltpu.async_copy(ws_hbm.at[pl.ds(r, 1)], rows_s.at[pl.ds(i, 1)], sem)
                pltpu.async_copy(wa_hbm.at[pl.ds(r, 1)], rows_a.at[pl.ds(i, 1)], sem)
            return carry

        lax.fori_loop(0, HALF // K, chunk_body, 0)
        pltpu.make_async_copy(ws_hbm.at[pl.ds(0, HALF)], rows_s, sem).wait()
        pltpu.make_async_copy(wa_hbm.at[pl.ds(0, HALF)], rows_a, sem).wait()
        pltpu.sync_copy(rows_s, out_s.at[pl.ds(base + off, HALF)])
        pltpu.sync_copy(rows_a, out_a.at[pl.ds(base + off, HALF)])


def kernel(instance_ids, W_shape, W_appearance):
    idx = instance_ids.astype(jnp.int32)
    return _gather2(idx, W_shape, W_appearance)
